# baseline (speedup 1.0000x reference)
import functools

import jax
import jax.numpy as jnp
import numpy as np
from jax import lax
from jax.experimental import pallas as pl
from jax.experimental.pallas import tpu as pltpu
from jax.experimental.pallas import tpu_sc as plsc

B = 10000
K = 32
U = 128
D_IN = 128

NC = 2
NS = 16
L = 16
NW = NC * NS
B_PAD = 10240
BPW = B_PAD // NW
APC = 4
ROWS = APC * K
NCHUNK = BPW // APC
NBUF = 2
OB = 2
HW = U // 2
NV = U // L
NH = U // (2 * L)
KU = 4
T_ROWS = 10240
SPS = 624
ZR = B


@functools.cache
def _make_sc_gather_sum():
    mesh = plsc.VectorSubcoreMesh(core_axis_name="c", subcore_axis_name="s")
    return functools.partial(
        pl.kernel,
        mesh=mesh,
        out_type=jax.ShapeDtypeStruct((B_PAD, U), jnp.bfloat16),
        scratch_types=[
            pltpu.VMEM((NCHUNK, ROWS), jnp.int32),
            pltpu.VMEM((NBUF, ROWS, HW), jnp.int32),
            pltpu.VMEM((OB, APC, U), jnp.bfloat16),
            pltpu.VMEM((8, HW), jnp.int32),
            pltpu.VMEM_SHARED((T_ROWS, HW), jnp.int32),
        ] + [pltpu.SemaphoreType.DMA] * (NBUF + OB),
    )(_sc_gather_sum_body)


def _sc_gather_sum_body(states_hbm, idx_hbm, out_hbm, idx_v, rows_v, outb_v,
                        zbuf_v, table_sh, *sems):
    gsems = sems[:NBUF]
    osems = sems[NBUF:]
    sid = lax.axis_index("s")
    wid = sid * NC + lax.axis_index("c")
    pltpu.sync_copy(states_hbm.at[pl.ds(sid * SPS, SPS)],
                    table_sh.at[pl.ds(sid * SPS, SPS)])

    @pl.when(sid == NS - 1)
    def _tail():
        pltpu.sync_copy(states_hbm.at[pl.ds(NS * SPS, B - NS * SPS)],
                        table_sh.at[pl.ds(NS * SPS, B - NS * SPS)])

    @pl.when(sid == 0)
    def _zero_row():
        for r in range(8):
            for j in range(HW // L):
                zbuf_v[r, pl.ds(j * L, L)] = jnp.zeros((L,), jnp.int32)
        pltpu.sync_copy(zbuf_v, table_sh.at[pl.ds(ZR, 8)])

    pltpu.sync_copy(idx_hbm.at[wid], idx_v)
    plsc.subcore_barrier()

    rows_b = rows_v.bitcast(jnp.bfloat16)

    for b in range(NBUF):
        pltpu.async_copy(table_sh.at[idx_v.at[b]], rows_v.at[b], gsems[b])

    obase = wid * BPW

    def step_body(step, _):
        for b in range(NBUF):
            c = step * NBUF + b
            ob = b
            pltpu.make_async_copy(
                table_sh.at[idx_v.at[c]], rows_v.at[b], gsems[b]
            ).wait()

            @pl.when(c >= OB)
            def _drain(_ob=ob):
                pltpu.make_async_copy(
                    outb_v.at[_ob], out_hbm.at[pl.ds(obase, APC)], osems[_ob]
                ).wait()

            for a in range(APC):
                accs = [jnp.zeros((2 * L,), jnp.bfloat16) for _ in range(NH)]
                for k in range(K):
                    r = a * K + k
                    for j in range(NH):
                        vr = 2 * r + j // 2
                        accs[j] = accs[j] + rows_b[
                            2 * b + vr // ROWS, vr % ROWS,
                            pl.ds((j % 2) * 2 * L, 2 * L)]
                for j in range(NH):
                    outb_v[ob, a, pl.ds(j * 2 * L, 2 * L)] = accs[j]
            pltpu.async_copy(
                outb_v.at[ob], out_hbm.at[pl.ds(obase + c * APC, APC)],
                osems[ob])
            nxt = c + NBUF

            @pl.when(nxt < NCHUNK)
            def _prefetch(_b=b, _nxt=nxt):
                pltpu.async_copy(
                    table_sh.at[idx_v.at[_nxt]], rows_v.at[_b], gsems[_b]
                )
        return 0

    lax.fori_loop(0, NCHUNK // NBUF, step_body, 0)
    for ob in range(OB):
        pltpu.make_async_copy(
            outb_v.at[ob], out_hbm.at[pl.ds(obase, APC)], osems[ob]
        ).wait()


BLK = 1000


def _tc_gru_body(x_ref, h_ref, s_ref, pi_ref, wx_ref, wf_ref, wr_ref, b_ref,
                 o_ref):
    cnt = jnp.sum((pi_ref[...] >= 0).astype(jnp.float32), axis=1,
                  keepdims=True)
    feat = s_ref[...].astype(jnp.float32) / (1e-5 + cnt)
    x = x_ref[...]
    h = h_ref[...]
    xm = (jnp.dot(x, wx_ref[...], preferred_element_type=jnp.float32)
          + jnp.dot(feat, wf_ref[...], preferred_element_type=jnp.float32)
          + b_ref[0:1, :])
    hm = (jnp.dot(h, wr_ref[...], preferred_element_type=jnp.float32)
          + b_ref[1:2, :])
    xz, xr, xh = xm[:, :U], xm[:, U:2 * U], xm[:, 2 * U:]
    hz, hr, hh = hm[:, :U], hm[:, U:2 * U], hm[:, 2 * U:]
    z = jax.nn.sigmoid(xz + hz)
    r = jax.nn.sigmoid(xr + hr)
    cand = jnp.tanh(xh + r * hh)
    o_ref[...] = z * h + (1.0 - z) * cand


def _tc_gru(x, h, sums, pidx, wx, wf, wr, bias):
    grid = (B // BLK,)
    return pl.pallas_call(
        _tc_gru_body,
        grid=grid,
        in_specs=[
            pl.BlockSpec((BLK, D_IN), lambda i: (i, 0)),
            pl.BlockSpec((BLK, U), lambda i: (i, 0)),
            pl.BlockSpec((BLK, U), lambda i: (i, 0)),
            pl.BlockSpec((BLK, K), lambda i: (i, 0)),
            pl.BlockSpec((D_IN, 3 * U), lambda i: (0, 0)),
            pl.BlockSpec((U, 3 * U), lambda i: (0, 0)),
            pl.BlockSpec((U, 3 * U), lambda i: (0, 0)),
            pl.BlockSpec((2, 3 * U), lambda i: (0, 0)),
        ],
        out_specs=pl.BlockSpec((BLK, U), lambda i: (i, 0)),
        out_shape=jax.ShapeDtypeStruct((B, U), jnp.float32),
    )(x, h, sums, pidx, wx, wf, wr, bias)


def kernel(inputs, rnn_states, kernel, recurrent_kernel, bias,
           present_indices):
    idx = jnp.where(present_indices < 0, ZR, present_indices)
    idx_pad = jnp.pad(idx, ((0, B_PAD - B), (0, 0)))
    idx3 = idx_pad.reshape(NW, NCHUNK, ROWS)
    states_i = lax.bitcast_convert_type(
        rnn_states.astype(jnp.bfloat16).reshape(B, HW, 2), jnp.int32)
    sums = _make_sc_gather_sum()(states_i, idx3)
    wx = kernel[:D_IN]
    wf = kernel[D_IN:]
    h_new = _tc_gru(inputs, rnn_states, sums, present_indices, wx, wf,
                    recurrent_kernel, bias)
    return (h_new, h_new)


# speedup vs baseline: 1.0426x; 1.0426x over previous
import functools

import jax
import jax.numpy as jnp
from jax import lax
from jax.experimental import pallas as pl
from jax.experimental.pallas import tpu as pltpu
from jax.experimental.pallas import tpu_sc as plsc

B = 10000
K = 32
U = 128
D_IN = 128

NC = 2
NS = 16
L = 16
NW = NC * NS
B_PAD = 10240
BPW = B_PAD // NW
APC = 4
ROWS = APC * K
NCHUNK = BPW // APC
NBUF = 2
OB = 2
NV = U // L
KU = 4
T_ROWS = 10240
SPS = 624
ZR = B


@functools.cache
def _make_sc_gather_sum():
    mesh = plsc.VectorSubcoreMesh(core_axis_name="c", subcore_axis_name="s")
    return functools.partial(
        pl.kernel,
        mesh=mesh,
        out_type=jax.ShapeDtypeStruct((B_PAD, U), jnp.float32),
        scratch_types=[
            pltpu.VMEM((NCHUNK, ROWS), jnp.int32),
            pltpu.VMEM((NBUF, ROWS, U), jnp.float32),
            pltpu.VMEM((OB, APC, U), jnp.float32),
            pltpu.VMEM((8, U), jnp.float32),
            pltpu.VMEM_SHARED((T_ROWS, U), jnp.float32),
        ] + [pltpu.SemaphoreType.DMA] * (NBUF + OB),
    )(_sc_gather_sum_body)


def _sc_gather_sum_body(states_hbm, idx_hbm, out_hbm, idx_v, rows_v, outb_v,
                        zbuf_v, table_sh, *sems):
    gsems = sems[:NBUF]
    osems = sems[NBUF:]
    sid = lax.axis_index("s")
    wid = sid * NC + lax.axis_index("c")
    pltpu.sync_copy(states_hbm.at[pl.ds(sid * SPS, SPS)],
                    table_sh.at[pl.ds(sid * SPS, SPS)])

    @pl.when(sid == NS - 1)
    def _tail():
        pltpu.sync_copy(states_hbm.at[pl.ds(NS * SPS, B - NS * SPS)],
                        table_sh.at[pl.ds(NS * SPS, B - NS * SPS)])

    @pl.when(sid == 0)
    def _zero_row():
        for r in range(8):
            for v in range(NV):
                zbuf_v[r, pl.ds(v * L, L)] = jnp.zeros((L,), jnp.float32)
        pltpu.sync_copy(zbuf_v, table_sh.at[pl.ds(ZR, 8)])

    pltpu.sync_copy(idx_hbm.at[wid], idx_v)
    plsc.subcore_barrier()

    for b in range(NBUF):
        pltpu.async_copy(table_sh.at[idx_v.at[b]], rows_v.at[b], gsems[b])

    obase = wid * BPW

    def step_body(step, _):
        for b in range(NBUF):
            c = step * NBUF + b
            ob = b
            pltpu.make_async_copy(
                table_sh.at[idx_v.at[c]], rows_v.at[b], gsems[b]
            ).wait()

            @pl.when(c >= OB)
            def _drain(_ob=ob):
                pltpu.make_async_copy(
                    outb_v.at[_ob], out_hbm.at[pl.ds(obase, APC)], osems[_ob]
                ).wait()

            for a in range(APC):
                def kbody(kk, accs, _a=a, _b=b):
                    base = _a * K + kk * KU
                    for u in range(KU):
                        accs = tuple(
                            accs[v] + rows_v[_b, base + u, pl.ds(v * L, L)]
                            for v in range(NV)
                        )
                    return accs
                accs = lax.fori_loop(
                    0, K // KU, kbody,
                    tuple(jnp.zeros((L,), jnp.float32) for _ in range(NV)),
                )
                for v in range(NV):
                    outb_v[ob, a, pl.ds(v * L, L)] = accs[v]
            pltpu.async_copy(
                outb_v.at[ob], out_hbm.at[pl.ds(obase + c * APC, APC)],
                osems[ob])
            nxt = c + NBUF

            @pl.when(nxt < NCHUNK)
            def _prefetch(_b=b, _nxt=nxt):
                pltpu.async_copy(
                    table_sh.at[idx_v.at[_nxt]], rows_v.at[_b], gsems[_b]
                )
        return 0

    lax.fori_loop(0, NCHUNK // NBUF, step_body, 0)
    for ob in range(OB):
        pltpu.make_async_copy(
            outb_v.at[ob], out_hbm.at[pl.ds(obase, APC)], osems[ob]
        ).wait()


BLK = 1000


def _tc_gru_body(x_ref, h_ref, s_ref, pi_ref, wx_ref, wf_ref, wr_ref, b_ref,
                 o_ref):
    cnt = jnp.sum((pi_ref[...] >= 0).astype(jnp.float32), axis=1,
                  keepdims=True)
    feat = s_ref[...] / (1e-5 + cnt)
    x = x_ref[...]
    h = h_ref[...]
    xm = (jnp.dot(x, wx_ref[...], preferred_element_type=jnp.float32)
          + jnp.dot(feat, wf_ref[...], preferred_element_type=jnp.float32)
          + b_ref[0:1, :])
    hm = (jnp.dot(h, wr_ref[...], preferred_element_type=jnp.float32)
          + b_ref[1:2, :])
    xz, xr, xh = xm[:, :U], xm[:, U:2 * U], xm[:, 2 * U:]
    hz, hr, hh = hm[:, :U], hm[:, U:2 * U], hm[:, 2 * U:]
    z = jax.nn.sigmoid(xz + hz)
    r = jax.nn.sigmoid(xr + hr)
    cand = jnp.tanh(xh + r * hh)
    o_ref[...] = z * h + (1.0 - z) * cand


def _tc_gru(x, h, sums, pidx, wx, wf, wr, bias):
    grid = (B // BLK,)
    return pl.pallas_call(
        _tc_gru_body,
        grid=grid,
        in_specs=[
            pl.BlockSpec((BLK, D_IN), lambda i: (i, 0)),
            pl.BlockSpec((BLK, U), lambda i: (i, 0)),
            pl.BlockSpec((BLK, U), lambda i: (i, 0)),
            pl.BlockSpec((BLK, K), lambda i: (i, 0)),
            pl.BlockSpec((D_IN, 3 * U), lambda i: (0, 0)),
            pl.BlockSpec((U, 3 * U), lambda i: (0, 0)),
            pl.BlockSpec((U, 3 * U), lambda i: (0, 0)),
            pl.BlockSpec((2, 3 * U), lambda i: (0, 0)),
        ],
        out_specs=pl.BlockSpec((BLK, U), lambda i: (i, 0)),
        out_shape=jax.ShapeDtypeStruct((B, U), jnp.float32),
    )(x, h, sums, pidx, wx, wf, wr, bias)


def kernel(inputs, rnn_states, kernel, recurrent_kernel, bias,
           present_indices):
    idx = jnp.where(present_indices < 0, ZR, present_indices)
    idx_pad = jnp.pad(idx, ((0, B_PAD - B), (0, 0)))
    idx3 = idx_pad.reshape(NW, NCHUNK, ROWS)
    sums = _make_sc_gather_sum()(rnn_states, idx3)
    wx = kernel[:D_IN]
    wf = kernel[D_IN:]
    h_new = _tc_gru(inputs, rnn_states, sums, present_indices, wx, wf,
                    recurrent_kernel, bias)
    return (h_new, h_new)


# speedup vs baseline: 1.0976x; 1.0528x over previous
import functools

import jax
import jax.numpy as jnp
from jax import lax
from jax.experimental import pallas as pl
from jax.experimental.pallas import tpu as pltpu
from jax.experimental.pallas import tpu_sc as plsc

B = 10000
K = 32
U = 128
D_IN = 128

NC = 2
NS = 16
L = 16
NW = NC * NS
B_PAD = 10240
BPW = B_PAD // NW
APC = 4
ROWS = APC * K
NCHUNK = BPW // APC
NBUF = 2
OB = 2
NV = U // L
KU = 4
T_ROWS = 10240
SPS = 624
ZR = B


@functools.cache
def _make_sc_gather_sum():
    mesh = plsc.VectorSubcoreMesh(core_axis_name="c", subcore_axis_name="s")
    return functools.partial(
        pl.kernel,
        mesh=mesh,
        out_type=jax.ShapeDtypeStruct((B_PAD, U), jnp.float32),
        scratch_types=[
            pltpu.VMEM((NCHUNK, ROWS), jnp.int32),
            pltpu.VMEM((NBUF, ROWS, U), jnp.float32),
            pltpu.VMEM((OB, APC, U), jnp.float32),
            pltpu.VMEM((8, U), jnp.float32),
            pltpu.VMEM_SHARED((T_ROWS, U), jnp.float32),
        ] + [pltpu.SemaphoreType.DMA] * (NBUF + OB),
    )(_sc_gather_sum_body)


def _sc_gather_sum_body(states_hbm, idx_hbm, out_hbm, idx_v, rows_v, outb_v,
                        zbuf_v, table_sh, *sems):
    gsems = sems[:NBUF]
    osems = sems[NBUF:]
    sid = lax.axis_index("s")
    wid = sid * NC + lax.axis_index("c")
    pltpu.sync_copy(states_hbm.at[pl.ds(sid * SPS, SPS)],
                    table_sh.at[pl.ds(sid * SPS, SPS)])

    @pl.when(sid == NS - 1)
    def _tail():
        pltpu.sync_copy(states_hbm.at[pl.ds(NS * SPS, B - NS * SPS)],
                        table_sh.at[pl.ds(NS * SPS, B - NS * SPS)])

    @pl.when(sid == 0)
    def _zero_row():
        for r in range(8):
            for v in range(NV):
                zbuf_v[r, pl.ds(v * L, L)] = jnp.zeros((L,), jnp.float32)
        pltpu.sync_copy(zbuf_v, table_sh.at[pl.ds(ZR, 8)])

    pltpu.sync_copy(idx_hbm.at[wid], idx_v)
    plsc.subcore_barrier()

    for b in range(NBUF):
        pltpu.async_copy(table_sh.at[idx_v.at[b]], rows_v.at[b], gsems[b])

    obase = wid * BPW

    def step_body(step, _):
        for b in range(NBUF):
            c = step * NBUF + b
            ob = b
            pltpu.make_async_copy(
                table_sh.at[idx_v.at[c]], rows_v.at[b], gsems[b]
            ).wait()

            @pl.when(c >= OB)
            def _drain(_ob=ob):
                pltpu.make_async_copy(
                    outb_v.at[_ob], out_hbm.at[pl.ds(obase, APC)], osems[_ob]
                ).wait()

            for a in range(APC):
                def kbody(kk, accs, _a=a, _b=b):
                    base = _a * K + kk * KU
                    for u in range(KU):
                        accs = tuple(
                            accs[v] + rows_v[_b, base + u, pl.ds(v * L, L)]
                            for v in range(NV)
                        )
                    return accs
                accs = lax.fori_loop(
                    0, K // KU, kbody,
                    tuple(jnp.zeros((L,), jnp.float32) for _ in range(NV)),
                )
                for v in range(NV):
                    outb_v[ob, a, pl.ds(v * L, L)] = accs[v]
            pltpu.async_copy(
                outb_v.at[ob], out_hbm.at[pl.ds(obase + c * APC, APC)],
                osems[ob])
            nxt = c + NBUF

            @pl.when(nxt < NCHUNK)
            def _prefetch(_b=b, _nxt=nxt):
                pltpu.async_copy(
                    table_sh.at[idx_v.at[_nxt]], rows_v.at[_b], gsems[_b]
                )
        return 0

    lax.fori_loop(0, NCHUNK // NBUF, step_body, 0)
    for ob in range(OB):
        pltpu.make_async_copy(
            outb_v.at[ob], out_hbm.at[pl.ds(obase, APC)], osems[ob]
        ).wait()


BLK = 2000


def _tc_gru_body(x_ref, h_ref, s_ref, pi_ref, wx_ref, wf_ref, wr_ref, b_ref,
                 o_ref, o2_ref):
    cnt = jnp.sum((pi_ref[...] >= 0).astype(jnp.float32), axis=1,
                  keepdims=True)
    feat = s_ref[...] / (1e-5 + cnt)
    x = x_ref[...]
    h = h_ref[...]
    xm = (jnp.dot(x, wx_ref[...], preferred_element_type=jnp.float32)
          + jnp.dot(feat, wf_ref[...], preferred_element_type=jnp.float32)
          + b_ref[0:1, :])
    hm = (jnp.dot(h, wr_ref[...], preferred_element_type=jnp.float32)
          + b_ref[1:2, :])
    xz, xr, xh = xm[:, :U], xm[:, U:2 * U], xm[:, 2 * U:]
    hz, hr, hh = hm[:, :U], hm[:, U:2 * U], hm[:, 2 * U:]
    z = jax.nn.sigmoid(xz + hz)
    r = jax.nn.sigmoid(xr + hr)
    cand = jnp.tanh(xh + r * hh)
    h_new = z * h + (1.0 - z) * cand
    o_ref[...] = h_new
    o2_ref[...] = h_new


def _tc_gru(x, h, sums, pidx, wx, wf, wr, bias):
    grid = (B // BLK,)
    return pl.pallas_call(
        _tc_gru_body,
        grid=grid,
        in_specs=[
            pl.BlockSpec((BLK, D_IN), lambda i: (i, 0)),
            pl.BlockSpec((BLK, U), lambda i: (i, 0)),
            pl.BlockSpec((BLK, U), lambda i: (i, 0)),
            pl.BlockSpec((BLK, K), lambda i: (i, 0)),
            pl.BlockSpec((D_IN, 3 * U), lambda i: (0, 0)),
            pl.BlockSpec((U, 3 * U), lambda i: (0, 0)),
            pl.BlockSpec((U, 3 * U), lambda i: (0, 0)),
            pl.BlockSpec((2, 3 * U), lambda i: (0, 0)),
        ],
        out_specs=[pl.BlockSpec((BLK, U), lambda i: (i, 0)),
                   pl.BlockSpec((BLK, U), lambda i: (i, 0))],
        out_shape=[jax.ShapeDtypeStruct((B, U), jnp.float32),
                   jax.ShapeDtypeStruct((B, U), jnp.float32)],
    )(x, h, sums, pidx, wx, wf, wr, bias)


def kernel(inputs, rnn_states, kernel, recurrent_kernel, bias,
           present_indices):
    idx = jnp.where(present_indices < 0, ZR, present_indices)
    idx_pad = jnp.pad(idx, ((0, B_PAD - B), (0, 0)))
    idx3 = idx_pad.reshape(NW, NCHUNK, ROWS)
    sums = _make_sc_gather_sum()(rnn_states, idx3)
    wx = kernel[:D_IN]
    wf = kernel[D_IN:]
    h_new, h_new2 = _tc_gru(inputs, rnn_states, sums, present_indices, wx, wf,
                            recurrent_kernel, bias)
    return (h_new, h_new2)


# speedup vs baseline: 1.1134x; 1.0144x over previous
import functools

import jax
import jax.numpy as jnp
from jax import lax
from jax.experimental import pallas as pl
from jax.experimental.pallas import tpu as pltpu
from jax.experimental.pallas import tpu_sc as plsc

B = 10000
K = 32
U = 128
D_IN = 128

NC = 2
NS = 16
L = 16
NW = NC * NS
B_PAD = 10240
BPW = B_PAD // NW
APC = 4
ROWS = APC * K
NCHUNK = BPW // APC
NBUF = 2
OB = 2
NV = U // L
KU = 4
T_ROWS = 10240
SPS = 624
ZR = B


@functools.cache
def _make_sc_gather_sum():
    mesh = plsc.VectorSubcoreMesh(core_axis_name="c", subcore_axis_name="s")
    return functools.partial(
        pl.kernel,
        mesh=mesh,
        out_type=jax.ShapeDtypeStruct((B_PAD, U), jnp.float32),
        scratch_types=[
            pltpu.VMEM((NCHUNK, ROWS), jnp.int32),
            pltpu.VMEM((NBUF, ROWS, U), jnp.float32),
            pltpu.VMEM((OB, APC, U), jnp.float32),
            pltpu.VMEM((8, U), jnp.float32),
            pltpu.VMEM_SHARED((T_ROWS, U), jnp.float32),
        ] + [pltpu.SemaphoreType.DMA] * (NBUF + OB),
    )(_sc_gather_sum_body)


def _sc_gather_sum_body(states_hbm, idx_hbm, out_hbm, idx_v, rows_v, outb_v,
                        zbuf_v, table_sh, *sems):
    gsems = sems[:NBUF]
    osems = sems[NBUF:]
    sid = lax.axis_index("s")
    wid = sid * NC + lax.axis_index("c")
    pltpu.sync_copy(states_hbm.at[pl.ds(sid * SPS, SPS)],
                    table_sh.at[pl.ds(sid * SPS, SPS)])

    @pl.when(sid == NS - 1)
    def _tail():
        pltpu.sync_copy(states_hbm.at[pl.ds(NS * SPS, B - NS * SPS)],
                        table_sh.at[pl.ds(NS * SPS, B - NS * SPS)])

    @pl.when(sid == 0)
    def _zero_row():
        for r in range(8):
            for v in range(NV):
                zbuf_v[r, pl.ds(v * L, L)] = jnp.zeros((L,), jnp.float32)
        pltpu.sync_copy(zbuf_v, table_sh.at[pl.ds(ZR, 8)])

    pltpu.sync_copy(idx_hbm.at[wid], idx_v)
    plsc.subcore_barrier()

    for b in range(NBUF):
        pltpu.async_copy(table_sh.at[idx_v.at[b]], rows_v.at[b], gsems[b])

    obase = wid * BPW

    def step_body(step, _):
        for b in range(NBUF):
            c = step * NBUF + b
            ob = b
            pltpu.make_async_copy(
                table_sh.at[idx_v.at[c]], rows_v.at[b], gsems[b]
            ).wait()

            @pl.when(c >= OB)
            def _drain(_ob=ob):
                pltpu.make_async_copy(
                    outb_v.at[_ob], out_hbm.at[pl.ds(obase, APC)], osems[_ob]
                ).wait()

            for a in range(APC):
                def kbody(kk, accs, _a=a, _b=b):
                    base = _a * K + kk * KU
                    for u in range(KU):
                        accs = tuple(
                            accs[v] + rows_v[_b, base + u, pl.ds(v * L, L)]
                            for v in range(NV)
                        )
                    return accs
                accs = lax.fori_loop(
                    0, K // KU, kbody,
                    tuple(jnp.zeros((L,), jnp.float32) for _ in range(NV)),
                )
                for v in range(NV):
                    outb_v[ob, a, pl.ds(v * L, L)] = accs[v]
            pltpu.async_copy(
                outb_v.at[ob], out_hbm.at[pl.ds(obase + c * APC, APC)],
                osems[ob])
            nxt = c + NBUF

            @pl.when(nxt < NCHUNK)
            def _prefetch(_b=b, _nxt=nxt):
                pltpu.async_copy(
                    table_sh.at[idx_v.at[_nxt]], rows_v.at[_b], gsems[_b]
                )
        return 0

    lax.fori_loop(0, NCHUNK // NBUF, step_body, 0)
    for ob in range(OB):
        pltpu.make_async_copy(
            outb_v.at[ob], out_hbm.at[pl.ds(obase, APC)], osems[ob]
        ).wait()


BLK = 2000


def _tc_gru_body(x_ref, h_ref, s_ref, pi_ref, wx_ref, wf_ref, wr_ref, b_ref,
                 o_ref, o2_ref):
    cnt = jnp.sum((pi_ref[...] >= 0).astype(jnp.float32), axis=1,
                  keepdims=True)
    feat = s_ref[...] / (1e-5 + cnt)
    x = x_ref[...]
    h = h_ref[...]
    xm = (jnp.dot(x, wx_ref[...], preferred_element_type=jnp.float32)
          + jnp.dot(feat, wf_ref[...], preferred_element_type=jnp.float32)
          + b_ref[0:1, :])
    hm = (jnp.dot(h, wr_ref[...], preferred_element_type=jnp.float32)
          + b_ref[1:2, :])
    xz, xr, xh = xm[:, :U], xm[:, U:2 * U], xm[:, 2 * U:]
    hz, hr, hh = hm[:, :U], hm[:, U:2 * U], hm[:, 2 * U:]
    z = jax.nn.sigmoid(xz + hz)
    r = jax.nn.sigmoid(xr + hr)
    cand = jnp.tanh(xh + r * hh)
    h_new = z * h + (1.0 - z) * cand
    o_ref[...] = h_new
    o2_ref[...] = h_new


def _tc_gru(x, h, sums, pidx, wx, wf, wr, bias):
    grid = (B // BLK,)
    return pl.pallas_call(
        _tc_gru_body,
        grid=grid,
        in_specs=[
            pl.BlockSpec((BLK, D_IN), lambda i: (i, 0)),
            pl.BlockSpec((BLK, U), lambda i: (i, 0)),
            pl.BlockSpec((BLK, U), lambda i: (i, 0)),
            pl.BlockSpec((BLK, K), lambda i: (i, 0)),
            pl.BlockSpec((D_IN, 3 * U), lambda i: (0, 0)),
            pl.BlockSpec((U, 3 * U), lambda i: (0, 0)),
            pl.BlockSpec((U, 3 * U), lambda i: (0, 0)),
            pl.BlockSpec((2, 3 * U), lambda i: (0, 0)),
        ],
        out_specs=[pl.BlockSpec((BLK, U), lambda i: (i, 0)),
                   pl.BlockSpec((BLK, U), lambda i: (i, 0))],
        out_shape=[jax.ShapeDtypeStruct((B, U), jnp.float32),
                   jax.ShapeDtypeStruct((B, U), jnp.float32)],
    )(x, h, sums, pidx, wx, wf, wr, bias)


def kernel(inputs, rnn_states, kernel, recurrent_kernel, bias,
           present_indices):
    idx_pad = jnp.pad(present_indices, ((0, B_PAD - B), (0, 0)))
    idx3 = jnp.where(idx_pad < 0, ZR, idx_pad).reshape(NW, NCHUNK, ROWS)
    sums = _make_sc_gather_sum()(rnn_states, idx3)
    wx = kernel[:D_IN]
    wf = kernel[D_IN:]
    h_new, h_new2 = _tc_gru(inputs, rnn_states, sums, present_indices, wx, wf,
                            recurrent_kernel, bias)
    return (h_new, h_new2)
